# baseline (speedup 1.0000x reference)
import jax
import jax.numpy as jnp
from jax.experimental import pallas as pl
from jax.experimental.pallas import tpu as pltpu

_CARD = 48
_NB_LABELS = 8
_B = 4
_P = 64

_N_NODE_PAIRS = _NB_LABELS * (_NB_LABELS - 1) // 2

_HI = jax.lax.Precision.HIGHEST


def _mm(a, b):
    return jnp.dot(a, b, precision=_HI, preferred_element_type=jnp.float32)


def _dg(a, b, dims):
    return jax.lax.dot_general(a, b, dimension_numbers=(dims, ((), ())),
                               precision=_HI, preferred_element_type=jnp.float32)


def _rb(x):
    return x.astype(jnp.bfloat16).astype(jnp.float32)


def _unfold(row):
    return jnp.concatenate(
        [row[i * _CARD:(i + 1) * _CARD][None, :] for i in range(_CARD)], axis=0)


def _ged_body(inp_ref, nw_ref, ew_ref, a_ref, lab_ref, o_ref, kscr, dscr, sscr,
              ascr, dma_sems):
    f32 = jnp.float32
    copies = []
    for k in range(_B):
        for t in range(2):
            cp = pltpu.make_async_copy(a_ref.at[inp_ref[k, t]],
                                       ascr.at[2 * k + t],
                                       dma_sems.at[2 * k + t])
            cp.start()
            copies.append(cp)
    aa = jax.lax.broadcasted_iota(jnp.int32, (_NB_LABELS, _NB_LABELS), 0)
    bb = jax.lax.broadcasted_iota(jnp.int32, (_NB_LABELS, _NB_LABELS), 1)
    lo = jnp.minimum(aa, bb)
    hi = jnp.maximum(aa, bb)
    pidx = lo * (2 * _NB_LABELS - lo - 1) // 2 + (hi - lo - 1)
    pidx = jnp.where(aa == bb, -1, pidx)
    nc8 = jnp.zeros((_NB_LABELS, _NB_LABELS), f32)
    for p in range(_N_NODE_PAIRS):
        nc8 = nc8 + jnp.maximum(nw_ref[p], 0.0) * (pidx == p).astype(f32)
    node_ins_del = jnp.maximum(nw_ref[_N_NODE_PAIRS], 0.0)
    ce01_b = _rb(jnp.maximum(ew_ref[0], 0.0))
    eid_b = _rb(jnp.maximum(ew_ref[1], 0.0))

    ii = jax.lax.broadcasted_iota(jnp.int32, (_P, _P), 0)
    jj = jax.lax.broadcasted_iota(jnp.int32, (_P, _P), 1)
    core = (ii < _CARD) & (jj < _CARD)
    valid = (ii <= _CARD) & (jj <= _CARD)
    corner = (ii == _CARD) & (jj == _CARD)
    sub8 = jax.lax.broadcasted_iota(jnp.int32, (_NB_LABELS, _CARD), 0)

    for k in range(_B):
        g1 = inp_ref[k, 0]
        g2 = inp_ref[k, 1]
        oh1t = (lab_ref[g1][None, :] == sub8).astype(f32)
        oh2t = (lab_ref[g2][None, :] == sub8).astype(f32)
        ncp = _dg(_dg(nc8, oh1t, ((0,), (0,))), oh2t, ((0,), (0,)))
        ncp64 = jnp.pad(ncp, ((0, _P - _CARD), (0, _P - _CARD)))
        dmat = jnp.where(core, ncp64, node_ins_del)
        dmat = jnp.where(corner, 0.0, dmat)
        dscr[k] = jnp.where(valid, dmat, 0.0)
        kscr[k] = jnp.where(valid, jnp.exp(-10.0 * dmat), 0.0)

    k3 = kscr[...]
    kb3 = _rb(k3)
    v3 = jnp.ones((_B, 1, _P), f32)
    for _ in range(5):
        u3 = 1.0 / (jnp.sum(kb3 * _rb(v3), axis=2, keepdims=True) + 1e-9)
        v3 = 1.0 / (jnp.sum(kb3 * _rb(u3), axis=1, keepdims=True) + 1e-9)
    s3 = u3 * k3 * v3
    sb3 = _rb(s3)
    r3 = jnp.sum(s3, axis=2, keepdims=True)
    rb3 = jnp.sum(sb3, axis=2, keepdims=True)
    ones3 = jnp.ones((_B, 1, _P), f32)
    c3 = jax.lax.dot_general(
        ones3, s3, dimension_numbers=(((2,), (1,)), ((0,), (0,))),
        precision=_HI, preferred_element_type=f32)
    cb3 = jax.lax.dot_general(
        ones3, sb3, dimension_numbers=(((2,), (1,)), ((0,), (0,))),
        precision=jax.lax.Precision.DEFAULT, preferred_element_type=f32)
    sscr[...] = s3

    for cp in copies:
        cp.wait()

    geds = []
    for k in range(_B):
        a1 = _unfold(ascr[2 * k])
        a2 = _unfold(ascr[2 * k + 1])
        s48 = sscr[k][:_CARD, :_CARD]
        sb48 = _rb(s48)
        r32 = r3[k][:_CARD, :]
        c32 = c3[k][:, :_CARD]
        rb_ = rb3[k][:_CARD, :]
        cb_ = cb3[k][:, :_CARD]
        b1 = (a1 != 0).astype(f32)
        b2 = (a2 != 0).astype(f32)
        t1 = jnp.sum(_mm(b1, r32) * rb_)
        t2 = jnp.sum(_dg(cb_, b2, ((1,), (1,))) * c32)
        t3 = jnp.sum(_mm(b1, _mm(s48, b2)) * sb48)
        e1a = (a1 == 1).astype(f32)
        e1b = (a1 == 2).astype(f32)
        e2a = (a2 == 1).astype(f32)
        e2b = (a2 == 2).astype(f32)
        t4 = jnp.sum(_mm(e1a, _mm(s48, e2b)) * sb48)
        t5 = jnp.sum(_mm(e1b, _mm(s48, e2a)) * sb48)
        quad = eid_b * (t1 + t2 - 2.0 * t3) + ce01_b * (t4 + t5)
        lin = jnp.sum(dscr[k] * sscr[k])
        geds.append(0.5 * quad + lin)

    gmin = jnp.minimum(jnp.minimum(geds[0], geds[1]), jnp.minimum(geds[2], geds[3]))
    gmax = jnp.maximum(jnp.maximum(geds[0], geds[1]), jnp.maximum(geds[2], geds[3]))
    inv = 1.0 / (gmax - gmin)
    lane = jax.lax.broadcasted_iota(jnp.int32, (_B,), 0)
    acc = jnp.zeros((_B,), f32)
    for k in range(_B):
        acc = acc + jnp.where(lane == k, (geds[k] - gmin) * inv, 0.0)
    o_ref[...] = acc


def kernel(input, node_weighs, edge_weighs, A, labels):
    return pl.pallas_call(
        _ged_body,
        out_shape=jax.ShapeDtypeStruct((_B,), jnp.float32),
        in_specs=[
            pl.BlockSpec(memory_space=pltpu.SMEM),
            pl.BlockSpec(memory_space=pltpu.SMEM),
            pl.BlockSpec(memory_space=pltpu.SMEM),
            pl.BlockSpec(memory_space=pltpu.MemorySpace.HBM),
            pl.BlockSpec(memory_space=pltpu.VMEM),
        ],
        out_specs=pl.BlockSpec(memory_space=pltpu.VMEM),
        scratch_shapes=[
            pltpu.VMEM((_B, _P, _P), jnp.float32),
            pltpu.VMEM((_B, _P, _P), jnp.float32),
            pltpu.VMEM((_B, _P, _P), jnp.float32),
            pltpu.VMEM((2 * _B, _CARD * _CARD), jnp.int32),
            pltpu.SemaphoreType.DMA((2 * _B,)),
        ],
    )(input, node_weighs, edge_weighs, A, labels)


# speedup vs baseline: 1.0069x; 1.0069x over previous
import jax
import jax.numpy as jnp
from jax.experimental import pallas as pl
from jax.experimental.pallas import tpu as pltpu

_CARD = 48
_NB_LABELS = 8
_B = 4
_P = 64

_N_NODE_PAIRS = _NB_LABELS * (_NB_LABELS - 1) // 2

_HI = jax.lax.Precision.HIGHEST


def _mm(a, b):
    return jnp.dot(a, b, precision=_HI, preferred_element_type=jnp.float32)


def _dg(a, b, dims):
    return jax.lax.dot_general(a, b, dimension_numbers=(dims, ((), ())),
                               precision=_HI, preferred_element_type=jnp.float32)


def _rb(x):
    return x.astype(jnp.bfloat16).astype(jnp.float32)


def _unfold(row):
    return jnp.concatenate(
        [row[i * _CARD:(i + 1) * _CARD][None, :] for i in range(_CARD)], axis=0)


def _ged_body(inp_ref, nw_ref, ew_ref, a_ref, lab_ref, o_ref, kscr, dscr, sscr):
    f32 = jnp.float32
    aa = jax.lax.broadcasted_iota(jnp.int32, (_NB_LABELS, _NB_LABELS), 0)
    bb = jax.lax.broadcasted_iota(jnp.int32, (_NB_LABELS, _NB_LABELS), 1)
    lo = jnp.minimum(aa, bb)
    hi = jnp.maximum(aa, bb)
    pidx = lo * (2 * _NB_LABELS - lo - 1) // 2 + (hi - lo - 1)
    pidx = jnp.where(aa == bb, -1, pidx)
    nc8 = jnp.zeros((_NB_LABELS, _NB_LABELS), f32)
    for p in range(_N_NODE_PAIRS):
        nc8 = nc8 + jnp.maximum(nw_ref[p], 0.0) * (pidx == p).astype(f32)
    node_ins_del = jnp.maximum(nw_ref[_N_NODE_PAIRS], 0.0)
    ce01_b = _rb(jnp.maximum(ew_ref[0], 0.0))
    eid_b = _rb(jnp.maximum(ew_ref[1], 0.0))

    ii = jax.lax.broadcasted_iota(jnp.int32, (_P, _P), 0)
    jj = jax.lax.broadcasted_iota(jnp.int32, (_P, _P), 1)
    core = (ii < _CARD) & (jj < _CARD)
    valid = (ii <= _CARD) & (jj <= _CARD)
    corner = (ii == _CARD) & (jj == _CARD)
    sub8 = jax.lax.broadcasted_iota(jnp.int32, (_NB_LABELS, _CARD), 0)

    for k in range(_B):
        g1 = inp_ref[k, 0]
        g2 = inp_ref[k, 1]
        oh1t = (lab_ref[g1][None, :] == sub8).astype(f32)
        oh2t = (lab_ref[g2][None, :] == sub8).astype(f32)
        ncp = _dg(_dg(nc8, oh1t, ((0,), (0,))), oh2t, ((0,), (0,)))
        ncp64 = jnp.pad(ncp, ((0, _P - _CARD), (0, _P - _CARD)))
        dmat = jnp.where(core, ncp64, node_ins_del)
        dmat = jnp.where(corner, 0.0, dmat)
        dscr[k] = jnp.where(valid, dmat, 0.0)
        kscr[k] = jnp.where(valid, jnp.exp(-10.0 * dmat), 0.0)

    k3 = kscr[...]
    kb3 = _rb(k3)
    v3 = jnp.ones((_B, 1, _P), f32)
    for _ in range(5):
        u3 = 1.0 / (jnp.sum(kb3 * _rb(v3), axis=2, keepdims=True) + 1e-9)
        v3 = 1.0 / (jnp.sum(kb3 * _rb(u3), axis=1, keepdims=True) + 1e-9)
    s3 = u3 * k3 * v3
    sb3 = _rb(s3)
    r3 = jnp.sum(s3, axis=2, keepdims=True)
    rb3 = jnp.sum(sb3, axis=2, keepdims=True)
    ones3 = jnp.ones((_B, 1, _P), f32)
    c3 = jax.lax.dot_general(
        ones3, s3, dimension_numbers=(((2,), (1,)), ((0,), (0,))),
        precision=_HI, preferred_element_type=f32)
    cb3 = jax.lax.dot_general(
        ones3, sb3, dimension_numbers=(((2,), (1,)), ((0,), (0,))),
        precision=jax.lax.Precision.DEFAULT, preferred_element_type=f32)
    sscr[...] = s3

    geds = []
    for k in range(_B):
        a1 = _unfold(a_ref[inp_ref[k, 0]])
        a2 = _unfold(a_ref[inp_ref[k, 1]])
        s48 = sscr[k][:_CARD, :_CARD]
        sb48 = _rb(s48)
        r32 = r3[k][:_CARD, :]
        c32 = c3[k][:, :_CARD]
        rb_ = rb3[k][:_CARD, :]
        cb_ = cb3[k][:, :_CARD]
        b1 = (a1 != 0).astype(f32)
        b2 = (a2 != 0).astype(f32)
        t1 = jnp.sum(_mm(b1, r32) * rb_)
        t2 = jnp.sum(_dg(cb_, b2, ((1,), (1,))) * c32)
        t3 = jnp.sum(_mm(b1, _mm(s48, b2)) * sb48)
        e1a = (a1 == 1).astype(f32)
        e1b = (a1 == 2).astype(f32)
        e2a = (a2 == 1).astype(f32)
        e2b = (a2 == 2).astype(f32)
        t4 = jnp.sum(_mm(e1a, _mm(s48, e2b)) * sb48)
        t5 = jnp.sum(_mm(e1b, _mm(s48, e2a)) * sb48)
        quad = eid_b * (t1 + t2 - 2.0 * t3) + ce01_b * (t4 + t5)
        lin = jnp.sum(dscr[k] * sscr[k])
        geds.append(0.5 * quad + lin)

    gmin = jnp.minimum(jnp.minimum(geds[0], geds[1]), jnp.minimum(geds[2], geds[3]))
    gmax = jnp.maximum(jnp.maximum(geds[0], geds[1]), jnp.maximum(geds[2], geds[3]))
    inv = 1.0 / (gmax - gmin)
    lane = jax.lax.broadcasted_iota(jnp.int32, (_B,), 0)
    acc = jnp.zeros((_B,), f32)
    for k in range(_B):
        acc = acc + jnp.where(lane == k, (geds[k] - gmin) * inv, 0.0)
    o_ref[...] = acc


def kernel(input, node_weighs, edge_weighs, A, labels):
    return pl.pallas_call(
        _ged_body,
        out_shape=jax.ShapeDtypeStruct((_B,), jnp.float32),
        in_specs=[
            pl.BlockSpec(memory_space=pltpu.SMEM),
            pl.BlockSpec(memory_space=pltpu.SMEM),
            pl.BlockSpec(memory_space=pltpu.SMEM),
            pl.BlockSpec(memory_space=pltpu.VMEM),
            pl.BlockSpec(memory_space=pltpu.VMEM),
        ],
        out_specs=pl.BlockSpec(memory_space=pltpu.VMEM),
        scratch_shapes=[
            pltpu.VMEM((_B, _P, _P), jnp.float32),
            pltpu.VMEM((_B, _P, _P), jnp.float32),
            pltpu.VMEM((_B, _P, _P), jnp.float32),
        ],
    )(input, node_weighs, edge_weighs, A, labels)
